# baseline (speedup 1.0000x reference)
import functools

import jax
import jax.numpy as jnp
import numpy as np
from jax import lax
from jax.experimental import pallas as pl
from jax.experimental.pallas import tpu as pltpu
from jax.experimental.pallas import tpu_sc as plsc

B = 4096
F = 26
V = 100000
D = 64
DENSE = 13
H1 = 1024
H2 = 512

_N = B * F
_Q = 128
_NC = 2
_NS = 16
_NW = _NC * _NS
_PER_W = _N // _NW
_CHUNK = 128
_NCHUNK = _PER_W // _CHUNK


def _gather_quad_rows(tabq, idxq):
    mesh = plsc.VectorSubcoreMesh(core_axis_name="c", subcore_axis_name="s")

    @functools.partial(
        pl.kernel,
        out_type=jax.ShapeDtypeStruct((_N, _Q), jnp.int32),
        mesh=mesh,
        scratch_types=[
            pltpu.VMEM((_PER_W,), jnp.int32),
            pltpu.VMEM((2, _CHUNK, _Q), jnp.int32),
            pltpu.SemaphoreType.DMA((2,)),
        ],
    )
    def gather_kernel(tab_hbm, idx_hbm, out_hbm, idx_v, buf, sem):
        wid = lax.axis_index("s") * _NC + lax.axis_index("c")
        base = wid * _PER_W
        pltpu.sync_copy(idx_hbm.at[pl.ds(base, _PER_W)], idx_v)

        pltpu.async_copy(
            tab_hbm.at[idx_v.at[pl.ds(0, _CHUNK)]], buf.at[0], sem.at[0]
        )

        def body(c, carry):
            slot = lax.rem(c, 2)
            nxt = lax.rem(c + 1, 2)

            @pl.when(c + 1 < _NCHUNK)
            def _():
                pltpu.async_copy(
                    tab_hbm.at[idx_v.at[pl.ds((c + 1) * _CHUNK, _CHUNK)]],
                    buf.at[nxt], sem.at[nxt],
                )

            pltpu.make_async_copy(
                tab_hbm.at[pl.ds(0, _CHUNK)], buf.at[slot], sem.at[slot]
            ).wait()
            pltpu.sync_copy(
                buf.at[slot], out_hbm.at[pl.ds(base + c * _CHUNK, _CHUNK)]
            )
            return carry

        lax.fori_loop(0, _NCHUNK, body, 0)

    return gather_kernel(tabq, idxq)


_BB = 512


def _mlp_body(x_ref, par_ref, dense_ref, sel_ref, w1e_ref, w1o_ref, w1d_ref,
              b1_ref, w2_ref, b2_ref, w3_ref, b3_ref, out_ref):
    q = jnp.dot(par_ref[...], sel_ref[...], preferred_element_type=jnp.float32)
    lane = lax.broadcasted_iota(jnp.int32, (1, F * _Q), 1)
    rowpat = ((lane % _Q) // 32).astype(jnp.float32)
    keep = q == rowpat
    xi = x_ref[...]
    xlo = lax.bitcast_convert_type(xi << 16, jnp.float32)
    xhi = lax.bitcast_convert_type(xi & jnp.int32(-65536), jnp.float32)
    xlo = jnp.where(keep, xlo, 0.0).astype(jnp.bfloat16)
    xhi = jnp.where(keep, xhi, 0.0).astype(jnp.bfloat16)
    h = jnp.dot(xlo, w1e_ref[...], preferred_element_type=jnp.float32)
    h = h + jnp.dot(xhi, w1o_ref[...], preferred_element_type=jnp.float32)
    h = h + jnp.dot(dense_ref[...], w1d_ref[...],
                    preferred_element_type=jnp.float32)
    h = jnp.maximum(h + b1_ref[...], 0.0).astype(jnp.bfloat16)
    h2 = jnp.dot(h, w2_ref[...], preferred_element_type=jnp.float32)
    h2 = jnp.maximum(h2 + b2_ref[...], 0.0)
    logit = jnp.sum(h2 * w3_ref[...], axis=1, keepdims=True) + b3_ref[...]
    out_ref[...] = jax.nn.sigmoid(logit)


def _mlp(x, par, dense, sel, W1e, W1o, W1d, b1, W2, b2, w3row, b3):
    return pl.pallas_call(
        _mlp_body,
        grid=(B // _BB,),
        in_specs=[
            pl.BlockSpec((_BB, F * _Q), lambda i: (i, 0)),
            pl.BlockSpec((_BB, F), lambda i: (i, 0)),
            pl.BlockSpec((_BB, DENSE), lambda i: (i, 0)),
            pl.BlockSpec((F, F * _Q), lambda i: (0, 0)),
            pl.BlockSpec((F * _Q, H1), lambda i: (0, 0)),
            pl.BlockSpec((F * _Q, H1), lambda i: (0, 0)),
            pl.BlockSpec((DENSE, H1), lambda i: (0, 0)),
            pl.BlockSpec((1, H1), lambda i: (0, 0)),
            pl.BlockSpec((H1, H2), lambda i: (0, 0)),
            pl.BlockSpec((1, H2), lambda i: (0, 0)),
            pl.BlockSpec((1, H2), lambda i: (0, 0)),
            pl.BlockSpec((1, 1), lambda i: (0, 0)),
        ],
        out_specs=pl.BlockSpec((_BB, 1), lambda i: (i, 0)),
        out_shape=jax.ShapeDtypeStruct((B, 1), jnp.float32),
    )(x, par, dense, sel, W1e, W1o, W1d, b1, W2, b2, w3row, b3)


_SEL = np.repeat(np.eye(F, dtype=np.float32), _Q, axis=1)


def kernel(sparse_ids, dense_feats, tables, W1, b1, W2, b2, W3, b3):
    tabq = lax.bitcast_convert_type(
        tables.astype(jnp.bfloat16).reshape(F * V // 4, _Q, 2), jnp.int32)
    offs = (jnp.arange(F, dtype=jnp.int32) * V)[None, :]
    flat_idx = sparse_ids.astype(jnp.int32) + offs
    idxq = (flat_idx >> 2).reshape(_N)
    sel4 = (flat_idx & 3).astype(jnp.float32)

    x = _gather_quad_rows(tabq, idxq).reshape(B, F * _Q)

    w1a = W1[:F * D].reshape(F, D, H1)
    w1e = jnp.broadcast_to(w1a[:, None, 0::2, :], (F, 4, 32, H1))
    W1e = w1e.reshape(F * _Q, H1).astype(jnp.bfloat16)
    w1o = jnp.broadcast_to(w1a[:, None, 1::2, :], (F, 4, 32, H1))
    W1o = w1o.reshape(F * _Q, H1).astype(jnp.bfloat16)
    W1d = W1[F * D:]
    W2b = W2.astype(jnp.bfloat16)

    return _mlp(x, sel4, dense_feats, _SEL, W1e, W1o, W1d,
                b1.reshape(1, H1), W2b, b2.reshape(1, H2),
                W3.reshape(1, H2), b3.reshape(1, 1))


# speedup vs baseline: 29.2262x; 29.2262x over previous
import functools

import jax
import jax.numpy as jnp
import numpy as np
from jax import lax
from jax.experimental import pallas as pl
from jax.experimental.pallas import tpu as pltpu
from jax.experimental.pallas import tpu_sc as plsc

B = 4096
F = 26
V = 100000
D = 64
DENSE = 13
H1 = 1024
H2 = 512

_N = B * F
_NE = _N * D
_NW = 32
_PER_W = _NE // _NW
_CHUNK = 8192
_NSTRM = _CHUNK // 128
_NCHUNK = _PER_W // _CHUNK


def _gather_elems(tab1, idxe):
    mesh = plsc.VectorSubcoreMesh(core_axis_name="c", subcore_axis_name="s")

    @functools.partial(
        pl.kernel,
        out_type=jax.ShapeDtypeStruct((_NE,), jnp.float32),
        mesh=mesh,
        scratch_types=[
            pltpu.VMEM((_CHUNK,), jnp.int32),
            pltpu.VMEM((_CHUNK,), jnp.float32),
            pltpu.SemaphoreType.DMA,
        ],
    )
    def gather_kernel(tab_hbm, idx_hbm, out_hbm, idx_v, buf, sem):
        wid = lax.axis_index("s") * 2 + lax.axis_index("c")
        base = wid * _PER_W

        def body(c, carry):
            off = base + c * _CHUNK
            pltpu.sync_copy(idx_hbm.at[pl.ds(off, _CHUNK)], idx_v)

            def strm(k, carryk):
                pltpu.async_copy(
                    tab_hbm.at[idx_v.at[pl.ds(k * 128, 128)]],
                    buf.at[pl.ds(k * 128, 128)],
                    sem,
                )
                return carryk

            lax.fori_loop(0, _NSTRM, strm, 0)
            pltpu.make_async_copy(
                tab_hbm.at[pl.ds(0, _CHUNK)], buf, sem
            ).wait()
            pltpu.sync_copy(buf, out_hbm.at[pl.ds(off, _CHUNK)])
            return carry

        lax.fori_loop(0, _NCHUNK, body, 0)

    return gather_kernel(tab1, idxe)


_BB = 512


def _mlp_body(x_ref, dense_ref, w1_ref, w1d_ref, b1_ref, w2_ref, b2_ref,
              w3_ref, b3_ref, out_ref):
    h = jnp.dot(x_ref[...].astype(jnp.bfloat16), w1_ref[...],
                preferred_element_type=jnp.float32)
    h = h + jnp.dot(dense_ref[...], w1d_ref[...],
                    preferred_element_type=jnp.float32)
    h = jnp.maximum(h + b1_ref[...], 0.0).astype(jnp.bfloat16)
    h2 = jnp.dot(h, w2_ref[...], preferred_element_type=jnp.float32)
    h2 = jnp.maximum(h2 + b2_ref[...], 0.0)
    logit = jnp.sum(h2 * w3_ref[...], axis=1, keepdims=True) + b3_ref[...]
    out_ref[...] = jax.nn.sigmoid(logit)


def _mlp(x, dense, W1a, W1d, b1, W2, b2, w3row, b3):
    return pl.pallas_call(
        _mlp_body,
        grid=(B // _BB,),
        in_specs=[
            pl.BlockSpec((_BB, F * D), lambda i: (i, 0)),
            pl.BlockSpec((_BB, DENSE), lambda i: (i, 0)),
            pl.BlockSpec((F * D, H1), lambda i: (0, 0)),
            pl.BlockSpec((DENSE, H1), lambda i: (0, 0)),
            pl.BlockSpec((1, H1), lambda i: (0, 0)),
            pl.BlockSpec((H1, H2), lambda i: (0, 0)),
            pl.BlockSpec((1, H2), lambda i: (0, 0)),
            pl.BlockSpec((1, H2), lambda i: (0, 0)),
            pl.BlockSpec((1, 1), lambda i: (0, 0)),
        ],
        out_specs=pl.BlockSpec((_BB, 1), lambda i: (i, 0)),
        out_shape=jax.ShapeDtypeStruct((B, 1), jnp.float32),
    )(x, dense, W1a, W1d, b1, W2, b2, w3row, b3)


def kernel(sparse_ids, dense_feats, tables, W1, b1, W2, b2, W3, b3):
    tab1 = tables.reshape(F * V * D)
    offs = (jnp.arange(F, dtype=jnp.int32) * V)[None, :]
    flat_idx = (sparse_ids.astype(jnp.int32) + offs).reshape(_N)
    idxe = (flat_idx[:, None] * D +
            jnp.arange(D, dtype=jnp.int32)[None, :]).reshape(_NE)

    x = _gather_elems(tab1, idxe).reshape(B, F * D)

    W1a = W1[:F * D].astype(jnp.bfloat16)
    W1d = W1[F * D:]
    W2b = W2.astype(jnp.bfloat16)

    return _mlp(x, dense_feats, W1a, W1d, b1.reshape(1, H1), W2b,
                b2.reshape(1, H2), W3.reshape(1, H2), b3.reshape(1, 1))


# speedup vs baseline: 36.5126x; 1.2493x over previous
import functools

import jax
import jax.numpy as jnp
import numpy as np
from jax import lax
from jax.experimental import pallas as pl
from jax.experimental.pallas import tpu as pltpu
from jax.experimental.pallas import tpu_sc as plsc

B = 4096
F = 26
V = 100000
D = 64
DENSE = 13
H1 = 1024
H2 = 512

_N = B * F
_NW = 32
_PER_W = _N // _NW
_CHUNK = 128
_NCHUNK = _PER_W // _CHUNK


def _gather_rows(flat_tables, flat_idx):
    mesh = plsc.VectorSubcoreMesh(core_axis_name="c", subcore_axis_name="s")

    @functools.partial(
        pl.kernel,
        out_type=jax.ShapeDtypeStruct((_N, D), jnp.float32),
        mesh=mesh,
        scratch_types=[
            pltpu.VMEM((_PER_W,), jnp.int32),
            pltpu.VMEM((2, _CHUNK, D), jnp.float32),
            pltpu.SemaphoreType.DMA((2,)),
        ],
        compiler_params=pltpu.CompilerParams(use_tc_tiling_on_sc=False),
    )
    def gather_kernel(tab_hbm, idx_hbm, out_hbm, idx_v, buf, sem):
        wid = lax.axis_index("s") * 2 + lax.axis_index("c")
        base = wid * _PER_W
        pltpu.sync_copy(idx_hbm.at[pl.ds(base, _PER_W)], idx_v)

        pltpu.async_copy(
            tab_hbm.at[idx_v.at[pl.ds(0, _CHUNK)]], buf.at[0], sem.at[0]
        )

        def body(c, carry):
            slot = lax.rem(c, 2)
            nxt = lax.rem(c + 1, 2)

            @pl.when(c + 1 < _NCHUNK)
            def _():
                pltpu.async_copy(
                    tab_hbm.at[idx_v.at[pl.ds((c + 1) * _CHUNK, _CHUNK)]],
                    buf.at[nxt], sem.at[nxt],
                )

            pltpu.make_async_copy(
                tab_hbm.at[pl.ds(0, _CHUNK)], buf.at[slot], sem.at[slot]
            ).wait()
            pltpu.sync_copy(
                buf.at[slot], out_hbm.at[pl.ds(base + c * _CHUNK, _CHUNK)]
            )
            return carry

        lax.fori_loop(0, _NCHUNK, body, 0)

    return gather_kernel(flat_tables, flat_idx)


_BB = 512


def _mlp_body(x_ref, dense_ref, w1_ref, w1d_ref, b1_ref, w2_ref, b2_ref,
              w3_ref, b3_ref, out_ref):
    h = jnp.dot(x_ref[...].astype(jnp.bfloat16), w1_ref[...],
                preferred_element_type=jnp.float32)
    h = h + jnp.dot(dense_ref[...], w1d_ref[...],
                    preferred_element_type=jnp.float32)
    h = jnp.maximum(h + b1_ref[...], 0.0).astype(jnp.bfloat16)
    h2 = jnp.dot(h, w2_ref[...], preferred_element_type=jnp.float32)
    h2 = jnp.maximum(h2 + b2_ref[...], 0.0)
    logit = jnp.sum(h2 * w3_ref[...], axis=1, keepdims=True) + b3_ref[...]
    out_ref[...] = jax.nn.sigmoid(logit)


def _mlp(x, dense, W1a, W1d, b1, W2, b2, w3row, b3):
    return pl.pallas_call(
        _mlp_body,
        grid=(B // _BB,),
        in_specs=[
            pl.BlockSpec((_BB, F * D), lambda i: (i, 0)),
            pl.BlockSpec((_BB, DENSE), lambda i: (i, 0)),
            pl.BlockSpec((F * D, H1), lambda i: (0, 0)),
            pl.BlockSpec((DENSE, H1), lambda i: (0, 0)),
            pl.BlockSpec((1, H1), lambda i: (0, 0)),
            pl.BlockSpec((H1, H2), lambda i: (0, 0)),
            pl.BlockSpec((1, H2), lambda i: (0, 0)),
            pl.BlockSpec((1, H2), lambda i: (0, 0)),
            pl.BlockSpec((1, 1), lambda i: (0, 0)),
        ],
        out_specs=pl.BlockSpec((_BB, 1), lambda i: (i, 0)),
        out_shape=jax.ShapeDtypeStruct((B, 1), jnp.float32),
    )(x, dense, W1a, W1d, b1, W2, b2, w3row, b3)


def kernel(sparse_ids, dense_feats, tables, W1, b1, W2, b2, W3, b3):
    flat_tables = tables.reshape(F * V, D)
    offs = (jnp.arange(F, dtype=jnp.int32) * V)[None, :]
    flat_idx = (sparse_ids.astype(jnp.int32) + offs).reshape(_N)

    x = _gather_rows(flat_tables, flat_idx).reshape(B, F * D)

    W1a = W1[:F * D].astype(jnp.bfloat16)
    W1d = W1[F * D:]
    W2b = W2.astype(jnp.bfloat16)

    return _mlp(x, dense_feats, W1a, W1d, b1.reshape(1, H1), W2b,
                b2.reshape(1, H2), W3.reshape(1, H2), b3.reshape(1, 1))
